# speedup vs baseline: 18.8636x; 18.8636x over previous
import functools

import jax
import jax.numpy as jnp
from jax import lax
from jax.experimental import pallas as pl
from jax.experimental.pallas import tpu as pltpu
from jax.experimental.pallas import tpu_sc as plsc

N = 10000
E = 320000
D = 128
G = 64
NC = 2
NS = 16
NW = NC * NS
EPW = E // NW
CH = 80
NCHUNK = EPW // CH
NPAD = 10240
RPT = NPAD // NS

_mesh = plsc.VectorSubcoreMesh(core_axis_name="c", subcore_axis_name="s")


@functools.partial(
    pl.kernel,
    out_type=jax.ShapeDtypeStruct((NC, NPAD, 16), jnp.float32),
    mesh=_mesh,
    scratch_types=[
        pltpu.VMEM((NCHUNK, CH), jnp.int32),
        pltpu.VMEM((CH, 16), jnp.float32),
        pltpu.VMEM_SHARED((NPAD, 16), jnp.float32),
        pltpu.SemaphoreType.DMA,
    ],
)
def _deg_kernel(dst_hbm, zeros_hbm, out_hbm, didx, ones_v, acc, sem):
    c = lax.axis_index("c")
    s = lax.axis_index("s")
    wid = s * NC + c

    def set_ones(i, carry):
        ones_v[i, :] = jnp.ones((16,), jnp.float32)
        return carry

    lax.fori_loop(0, CH, set_ones, 0)
    pltpu.sync_copy(dst_hbm.at[wid], didx)
    pltpu.sync_copy(zeros_hbm.at[pl.ds(s * RPT, RPT)], acc.at[pl.ds(s * RPT, RPT)])
    plsc.subcore_barrier()

    def body(i, carry):
        pltpu.sync_copy(ones_v, acc.at[didx.at[i]], add=True)
        return carry

    lax.fori_loop(0, NCHUNK, body, 0)
    plsc.subcore_barrier()
    pltpu.sync_copy(acc.at[pl.ds(s * RPT, RPT)], out_hbm.at[c, pl.ds(s * RPT, RPT)])


@functools.partial(
    pl.kernel,
    out_type=jax.ShapeDtypeStruct((NC, NPAD, D), jnp.float32),
    mesh=_mesh,
    scratch_types=[
        pltpu.VMEM((NCHUNK, CH), jnp.int32),
        pltpu.VMEM((NCHUNK, CH), jnp.int32),
        pltpu.VMEM((CH, D), jnp.float32),
        pltpu.VMEM_SHARED((NPAD, D), jnp.float32),
        pltpu.SemaphoreType.DMA,
    ],
)
def _agg_kernel(src_hbm, dst_hbm, g_hbm, zeros_hbm, out_hbm,
                sidx, didx, rows, acc, sem):
    c = lax.axis_index("c")
    s = lax.axis_index("s")
    wid = s * NC + c
    pltpu.sync_copy(src_hbm.at[wid], sidx)
    pltpu.sync_copy(dst_hbm.at[wid], didx)
    pltpu.sync_copy(zeros_hbm.at[pl.ds(s * RPT, RPT)], acc.at[pl.ds(s * RPT, RPT)])
    plsc.subcore_barrier()

    def body(i, carry):
        pltpu.async_copy(g_hbm.at[sidx.at[i]], rows, sem).wait()
        pltpu.sync_copy(rows, acc.at[didx.at[i]], add=True)
        return carry

    lax.fori_loop(0, NCHUNK, body, 0)
    plsc.subcore_barrier()
    pltpu.sync_copy(acc.at[pl.ds(s * RPT, RPT)], out_hbm.at[c, pl.ds(s * RPT, RPT)])


def _tc1_body(x_ref, w1_ref, d0_ref, d1_ref, h1_ref, g1_ref, dis_ref):
    deg = 1.0 + d0_ref[:, 0:1] + d1_ref[:, 0:1]
    dis = jnp.broadcast_to(lax.rsqrt(deg), (N, D))
    h1 = jnp.dot(x_ref[:, :], w1_ref[:, :], preferred_element_type=jnp.float32)
    h1_ref[:, :] = h1
    g1_ref[:, :] = dis * h1
    dis_ref[:, :] = dis


def _tc2_body(a0_ref, a1_ref, dis_ref, h1_ref, b1_ref, w2_ref, h2_ref, g2_ref):
    d = dis_ref[:, :]
    h1o = jnp.maximum(
        d * (a0_ref[:, :] + a1_ref[:, :]) + d * d * h1_ref[:, :] + b1_ref[:, :],
        0.0)
    h2 = jnp.dot(h1o, w2_ref[:, :], preferred_element_type=jnp.float32)
    h2_ref[:, :] = h2
    g2_ref[:, :] = d * h2


def _tc3_body(a0_ref, a1_ref, dis_ref, h2_ref, b2_ref, batch_ref, wfc_ref,
              bfc_ref, out_ref):
    d = dis_ref[:, :]
    h2o = jnp.maximum(
        d * (a0_ref[:, :] + a1_ref[:, :]) + d * d * h2_ref[:, :] + b2_ref[:, :],
        0.0)
    gids = lax.broadcasted_iota(jnp.int32, (G, N), 0)
    onehot_t = (gids == batch_ref[:, :]).astype(jnp.float32)
    sums = jnp.dot(onehot_t, h2o, preferred_element_type=jnp.float32)
    cnts = jnp.sum(onehot_t, axis=1, keepdims=True)
    emb = sums / jnp.maximum(cnts, 1.0)
    out_ref[:, :] = (
        jnp.dot(emb, wfc_ref[:, :], preferred_element_type=jnp.float32)
        + bfc_ref[:, :])


_tc1 = pl.pallas_call(
    _tc1_body,
    out_shape=[
        jax.ShapeDtypeStruct((N, D), jnp.float32),
        jax.ShapeDtypeStruct((N, D), jnp.float32),
        jax.ShapeDtypeStruct((N, D), jnp.float32),
    ],
)

_tc2 = pl.pallas_call(
    _tc2_body,
    out_shape=[
        jax.ShapeDtypeStruct((N, D), jnp.float32),
        jax.ShapeDtypeStruct((N, D), jnp.float32),
    ],
)

_tc3 = pl.pallas_call(
    _tc3_body,
    out_shape=jax.ShapeDtypeStruct((G, D), jnp.float32),
)


def kernel(x, edge_index, batch, W1, b1, W2, b2, Wfc, bfc):
    src = edge_index[0].reshape(NW, NCHUNK, CH)
    dst = edge_index[1].reshape(NW, NCHUNK, CH)
    zeros16 = jnp.zeros((NPAD, 16), jnp.float32)
    zeros128 = jnp.zeros((NPAD, D), jnp.float32)

    degp = _deg_kernel(dst, zeros16)
    h1, g1, dis_b = _tc1(x, W1, degp[0, :N], degp[1, :N])

    accp1 = _agg_kernel(src, dst, g1, zeros128)
    h2, g2 = _tc2(accp1[0, :N], accp1[1, :N], dis_b, h1,
                  b1.reshape(1, D), W2)

    accp2 = _agg_kernel(src, dst, g2, zeros128)
    out = _tc3(accp2[0, :N], accp2[1, :N], dis_b, h2,
               b2.reshape(1, D), batch.reshape(1, N), Wfc,
               bfc.reshape(1, D))
    return out


# speedup vs baseline: 24.1188x; 1.2786x over previous
import functools

import jax
import jax.numpy as jnp
from jax import lax
from jax.experimental import pallas as pl
from jax.experimental.pallas import tpu as pltpu
from jax.experimental.pallas import tpu_sc as plsc

N = 10000
E = 320000
D = 128
G = 64
NC = 2
NS = 16
NW = NC * NS
EPW = E // NW
CH = 50
NCHUNK = EPW // CH
SEC = 40
NSEC = NCHUNK // SEC
NPAD = 10112
RPT = NPAD // NS

_mesh = plsc.VectorSubcoreMesh(core_axis_name="c", subcore_axis_name="s")


@functools.partial(
    pl.kernel,
    out_type=jax.ShapeDtypeStruct((NC, NPAD, 16), jnp.float32),
    mesh=_mesh,
    scratch_types=[
        pltpu.VMEM((NCHUNK, CH), jnp.int32),
        pltpu.VMEM((CH, 16), jnp.float32),
        pltpu.VMEM_SHARED((NPAD, 16), jnp.float32),
        pltpu.SemaphoreType.DMA,
    ],
)
def _deg_kernel(dst_hbm, zeros_hbm, out_hbm, didx, ones_v, acc, sem):
    c = lax.axis_index("c")
    s = lax.axis_index("s")
    wid = s * NC + c

    def set_ones(i, carry):
        ones_v[i, :] = jnp.ones((16,), jnp.float32)
        return carry

    lax.fori_loop(0, CH, set_ones, 0)
    pltpu.sync_copy(dst_hbm.at[wid], didx)
    pltpu.sync_copy(zeros_hbm.at[pl.ds(s * RPT, RPT)], acc.at[pl.ds(s * RPT, RPT)])
    plsc.subcore_barrier()

    def body(i, carry):
        pltpu.sync_copy(ones_v, acc.at[didx.at[i]], add=True)
        return carry

    lax.fori_loop(0, NCHUNK, body, 0)
    plsc.subcore_barrier()
    pltpu.sync_copy(acc.at[pl.ds(s * RPT, RPT)], out_hbm.at[c, pl.ds(s * RPT, RPT)])


@functools.partial(
    pl.kernel,
    out_type=jax.ShapeDtypeStruct((NC, NPAD, D), jnp.float32),
    mesh=_mesh,
    scratch_types=[
        pltpu.VMEM((2, SEC, CH), jnp.int32),
        pltpu.VMEM((2, SEC, CH), jnp.int32),
        pltpu.VMEM((2, CH, D), jnp.float32),
        pltpu.VMEM_SHARED((NPAD, D), jnp.float32),
        pltpu.SemaphoreType.DMA,
        pltpu.SemaphoreType.DMA,
        pltpu.SemaphoreType.DMA,
        pltpu.SemaphoreType.DMA,
    ],
)
def _agg_kernel(src_hbm, dst_hbm, g_hbm, zeros_hbm, out_hbm,
                sidx, didx, rows, acc, sem0, sem1, isems, isemd):
    c = lax.axis_index("c")
    s = lax.axis_index("s")
    wid = s * NC + c
    pltpu.sync_copy(src_hbm.at[wid, 0], sidx.at[0])
    pltpu.sync_copy(dst_hbm.at[wid, 0], didx.at[0])
    pltpu.sync_copy(zeros_hbm.at[pl.ds(s * RPT, RPT)], acc.at[pl.ds(s * RPT, RPT)])
    plsc.subcore_barrier()

    pltpu.async_copy(g_hbm.at[sidx.at[0, 0]], rows.at[0], sem0)

    def gather(i, buf, sem):
        sec = i // SEC
        pltpu.async_copy(g_hbm.at[sidx.at[sec % 2, i % SEC]], rows.at[buf], sem)

    def gwait(i, buf, sem):
        sec = i // SEC
        pltpu.make_async_copy(
            g_hbm.at[sidx.at[sec % 2, i % SEC]], rows.at[buf], sem).wait()

    def scatter(i, buf):
        sec = i // SEC
        pltpu.sync_copy(rows.at[buf], acc.at[didx.at[sec % 2, i % SEC]],
                        add=True)

    def body(j, carry):
        i0 = 2 * j
        i1 = i0 + 1
        i2 = i0 + 2
        sec0 = i0 // SEC

        @pl.when((i0 % SEC == 0) & (sec0 + 1 < NSEC))
        def _():
            nxt = sec0 + 1
            pltpu.async_copy(src_hbm.at[wid, nxt], sidx.at[nxt % 2], isems)
            pltpu.async_copy(dst_hbm.at[wid, nxt], didx.at[nxt % 2], isemd)

        @pl.when((i2 < NCHUNK) & (i2 % SEC == 0))
        def _():
            nxt = i2 // SEC
            pltpu.make_async_copy(src_hbm.at[wid, nxt], sidx.at[nxt % 2],
                                  isems).wait()
            pltpu.make_async_copy(dst_hbm.at[wid, nxt], didx.at[nxt % 2],
                                  isemd).wait()

        @pl.when(i1 < NCHUNK)
        def _():
            gather(i1, 1, sem1)

        gwait(i0, 0, sem0)
        scatter(i0, 0)

        @pl.when(i2 < NCHUNK)
        def _():
            gather(i2, 0, sem0)

        @pl.when(i1 < NCHUNK)
        def _():
            gwait(i1, 1, sem1)
            scatter(i1, 1)

        return carry

    lax.fori_loop(0, (NCHUNK + 1) // 2, body, 0)
    plsc.subcore_barrier()
    pltpu.sync_copy(acc.at[pl.ds(s * RPT, RPT)], out_hbm.at[c, pl.ds(s * RPT, RPT)])


def _tc1_body(x_ref, w1_ref, d0_ref, d1_ref, h1_ref, g1_ref, dis_ref):
    deg = 1.0 + d0_ref[:, 0:1] + d1_ref[:, 0:1]
    dis = jnp.broadcast_to(lax.rsqrt(deg), (N, D))
    h1 = jnp.dot(x_ref[:, :], w1_ref[:, :], preferred_element_type=jnp.float32)
    h1_ref[:, :] = h1
    g1_ref[:, :] = dis * h1
    dis_ref[:, :] = dis


def _tc2_body(a0_ref, a1_ref, dis_ref, h1_ref, b1_ref, w2_ref, h2_ref, g2_ref):
    d = dis_ref[:, :]
    h1o = jnp.maximum(
        d * (a0_ref[:, :] + a1_ref[:, :]) + d * d * h1_ref[:, :] + b1_ref[:, :],
        0.0)
    h2 = jnp.dot(h1o, w2_ref[:, :], preferred_element_type=jnp.float32)
    h2_ref[:, :] = h2
    g2_ref[:, :] = d * h2


def _tc3_body(a0_ref, a1_ref, dis_ref, h2_ref, b2_ref, batch_ref, wfc_ref,
              bfc_ref, out_ref):
    d = dis_ref[:, :]
    h2o = jnp.maximum(
        d * (a0_ref[:, :] + a1_ref[:, :]) + d * d * h2_ref[:, :] + b2_ref[:, :],
        0.0)
    gids = lax.broadcasted_iota(jnp.int32, (G, N), 0)
    onehot_t = (gids == batch_ref[:, :]).astype(jnp.float32)
    sums = jnp.dot(onehot_t, h2o, preferred_element_type=jnp.float32)
    cnts = jnp.sum(onehot_t, axis=1, keepdims=True)
    emb = sums / jnp.maximum(cnts, 1.0)
    out_ref[:, :] = (
        jnp.dot(emb, wfc_ref[:, :], preferred_element_type=jnp.float32)
        + bfc_ref[:, :])


_tc1 = pl.pallas_call(
    _tc1_body,
    out_shape=[
        jax.ShapeDtypeStruct((N, D), jnp.float32),
        jax.ShapeDtypeStruct((N, D), jnp.float32),
        jax.ShapeDtypeStruct((N, D), jnp.float32),
    ],
)

_tc2 = pl.pallas_call(
    _tc2_body,
    out_shape=[
        jax.ShapeDtypeStruct((N, D), jnp.float32),
        jax.ShapeDtypeStruct((N, D), jnp.float32),
    ],
)

_tc3 = pl.pallas_call(
    _tc3_body,
    out_shape=jax.ShapeDtypeStruct((G, D), jnp.float32),
)


def kernel(x, edge_index, batch, W1, b1, W2, b2, Wfc, bfc):
    src = edge_index[0].reshape(NW, NSEC, SEC, CH)
    dst = edge_index[1].reshape(NW, NSEC, SEC, CH)
    dst_flat = edge_index[1].reshape(NW, NCHUNK, CH)
    zeros16 = jnp.zeros((NPAD, 16), jnp.float32)
    zeros128 = jnp.zeros((NPAD, D), jnp.float32)

    degp = _deg_kernel(dst_flat, zeros16)
    h1, g1, dis_b = _tc1(x, W1, degp[0, :N], degp[1, :N])

    accp1 = _agg_kernel(src, dst, g1, zeros128)
    h2, g2 = _tc2(accp1[0, :N], accp1[1, :N], dis_b, h1,
                  b1.reshape(1, D), W2)

    accp2 = _agg_kernel(src, dst, g2, zeros128)
    out = _tc3(accp2[0, :N], accp2[1, :N], dis_b, h2,
               b2.reshape(1, D), batch.reshape(1, N), Wfc,
               bfc.reshape(1, D))
    return out


# speedup vs baseline: 29.5430x; 1.2249x over previous
import functools

import jax
import jax.numpy as jnp
from jax import lax
from jax.experimental import pallas as pl
from jax.experimental.pallas import tpu as pltpu
from jax.experimental.pallas import tpu_sc as plsc

N = 10000
E = 320000
D = 128
G = 64
NC = 2
NS = 16
NW = NC * NS
EPW = E // NW
CH = 100
NCHUNK = EPW // CH
SEC = 20
NSEC = NCHUNK // SEC
NPAD = 10112
RPT = NPAD // NS

_mesh = plsc.VectorSubcoreMesh(core_axis_name="c", subcore_axis_name="s")


@functools.partial(
    pl.kernel,
    out_type=jax.ShapeDtypeStruct((NC, NPAD, 16), jnp.float32),
    mesh=_mesh,
    scratch_types=[
        pltpu.VMEM((NCHUNK, CH), jnp.int32),
        pltpu.VMEM((CH, 16), jnp.float32),
        pltpu.VMEM_SHARED((NPAD, 16), jnp.float32),
        pltpu.SemaphoreType.DMA,
    ],
)
def _deg_kernel(dst_hbm, zeros_hbm, out_hbm, didx, ones_v, acc, sem):
    c = lax.axis_index("c")
    s = lax.axis_index("s")
    wid = s * NC + c

    def set_ones(i, carry):
        ones_v[i, :] = jnp.ones((16,), jnp.float32)
        return carry

    lax.fori_loop(0, CH, set_ones, 0)
    pltpu.sync_copy(dst_hbm.at[wid], didx)
    pltpu.sync_copy(zeros_hbm.at[pl.ds(s * RPT, RPT)], acc.at[pl.ds(s * RPT, RPT)])
    plsc.subcore_barrier()

    def body(i, carry):
        pltpu.sync_copy(ones_v, acc.at[didx.at[i]], add=True)
        return carry

    lax.fori_loop(0, NCHUNK, body, 0)
    plsc.subcore_barrier()
    pltpu.sync_copy(acc.at[pl.ds(s * RPT, RPT)], out_hbm.at[c, pl.ds(s * RPT, RPT)])


@functools.partial(
    pl.kernel,
    out_type=jax.ShapeDtypeStruct((NC, NPAD, D), jnp.float32),
    mesh=_mesh,
    scratch_types=[
        pltpu.VMEM((2, SEC, CH), jnp.int32),
        pltpu.VMEM((2, SEC, CH), jnp.int32),
        pltpu.VMEM((2, CH, D), jnp.float32),
        pltpu.VMEM_SHARED((NPAD, D), jnp.float32),
        pltpu.SemaphoreType.DMA,
        pltpu.SemaphoreType.DMA,
        pltpu.SemaphoreType.DMA,
        pltpu.SemaphoreType.DMA,
    ],
)
def _agg_kernel(src_hbm, dst_hbm, g_hbm, zeros_hbm, out_hbm,
                sidx, didx, rows, acc, sem0, sem1, isems, isemd):
    c = lax.axis_index("c")
    s = lax.axis_index("s")
    wid = s * NC + c
    pltpu.sync_copy(src_hbm.at[wid, 0], sidx.at[0])
    pltpu.sync_copy(dst_hbm.at[wid, 0], didx.at[0])
    pltpu.sync_copy(zeros_hbm.at[pl.ds(s * RPT, RPT)], acc.at[pl.ds(s * RPT, RPT)])
    plsc.subcore_barrier()

    pltpu.async_copy(g_hbm.at[sidx.at[0, 0]], rows.at[0], sem0)

    def gather(i, buf, sem):
        sec = i // SEC
        pltpu.async_copy(g_hbm.at[sidx.at[sec % 2, i % SEC]], rows.at[buf], sem)

    def gwait(i, buf, sem):
        sec = i // SEC
        pltpu.make_async_copy(
            g_hbm.at[sidx.at[sec % 2, i % SEC]], rows.at[buf], sem).wait()

    def scatter(i, buf):
        sec = i // SEC
        pltpu.sync_copy(rows.at[buf], acc.at[didx.at[sec % 2, i % SEC]],
                        add=True)

    def body(j, carry):
        i0 = 2 * j
        i1 = i0 + 1
        i2 = i0 + 2
        sec0 = i0 // SEC

        @pl.when((i0 % SEC == 0) & (sec0 + 1 < NSEC))
        def _():
            nxt = sec0 + 1
            pltpu.async_copy(src_hbm.at[wid, nxt], sidx.at[nxt % 2], isems)
            pltpu.async_copy(dst_hbm.at[wid, nxt], didx.at[nxt % 2], isemd)

        @pl.when((i2 < NCHUNK) & (i2 % SEC == 0))
        def _():
            nxt = i2 // SEC
            pltpu.make_async_copy(src_hbm.at[wid, nxt], sidx.at[nxt % 2],
                                  isems).wait()
            pltpu.make_async_copy(dst_hbm.at[wid, nxt], didx.at[nxt % 2],
                                  isemd).wait()

        @pl.when(i1 < NCHUNK)
        def _():
            gather(i1, 1, sem1)

        gwait(i0, 0, sem0)
        scatter(i0, 0)

        @pl.when(i2 < NCHUNK)
        def _():
            gather(i2, 0, sem0)

        @pl.when(i1 < NCHUNK)
        def _():
            gwait(i1, 1, sem1)
            scatter(i1, 1)

        return carry

    lax.fori_loop(0, (NCHUNK + 1) // 2, body, 0)
    plsc.subcore_barrier()
    pltpu.sync_copy(acc.at[pl.ds(s * RPT, RPT)], out_hbm.at[c, pl.ds(s * RPT, RPT)])


def _tc1_body(x_ref, w1_ref, d0_ref, d1_ref, h1_ref, g1_ref, dis_ref):
    deg = 1.0 + d0_ref[:, 0:1] + d1_ref[:, 0:1]
    dis = jnp.broadcast_to(lax.rsqrt(deg), (N, D))
    h1 = jnp.dot(x_ref[:, :], w1_ref[:, :], preferred_element_type=jnp.float32)
    h1_ref[:, :] = h1
    g1_ref[:, :] = dis * h1
    dis_ref[:, :] = dis


def _tc2_body(a0_ref, a1_ref, dis_ref, h1_ref, b1_ref, w2_ref, h2_ref, g2_ref):
    d = dis_ref[:, :]
    h1o = jnp.maximum(
        d * (a0_ref[:, :] + a1_ref[:, :]) + d * d * h1_ref[:, :] + b1_ref[:, :],
        0.0)
    h2 = jnp.dot(h1o, w2_ref[:, :], preferred_element_type=jnp.float32)
    h2_ref[:, :] = h2
    g2_ref[:, :] = d * h2


def _tc3_body(a0_ref, a1_ref, dis_ref, h2_ref, b2_ref, batch_ref, wfc_ref,
              bfc_ref, out_ref):
    d = dis_ref[:, :]
    h2o = jnp.maximum(
        d * (a0_ref[:, :] + a1_ref[:, :]) + d * d * h2_ref[:, :] + b2_ref[:, :],
        0.0)
    gids = lax.broadcasted_iota(jnp.int32, (G, N), 0)
    onehot_t = (gids == batch_ref[:, :]).astype(jnp.float32)
    sums = jnp.dot(onehot_t, h2o, preferred_element_type=jnp.float32)
    cnts = jnp.sum(onehot_t, axis=1, keepdims=True)
    emb = sums / jnp.maximum(cnts, 1.0)
    out_ref[:, :] = (
        jnp.dot(emb, wfc_ref[:, :], preferred_element_type=jnp.float32)
        + bfc_ref[:, :])


_tc1 = pl.pallas_call(
    _tc1_body,
    out_shape=[
        jax.ShapeDtypeStruct((N, D), jnp.float32),
        jax.ShapeDtypeStruct((N, D), jnp.float32),
        jax.ShapeDtypeStruct((N, D), jnp.float32),
    ],
)

_tc2 = pl.pallas_call(
    _tc2_body,
    out_shape=[
        jax.ShapeDtypeStruct((N, D), jnp.float32),
        jax.ShapeDtypeStruct((N, D), jnp.float32),
    ],
)

_tc3 = pl.pallas_call(
    _tc3_body,
    out_shape=jax.ShapeDtypeStruct((G, D), jnp.float32),
)


def kernel(x, edge_index, batch, W1, b1, W2, b2, Wfc, bfc):
    src = edge_index[0].reshape(NW, NSEC, SEC, CH)
    dst = edge_index[1].reshape(NW, NSEC, SEC, CH)
    dst_flat = edge_index[1].reshape(NW, NCHUNK, CH)
    zeros16 = jnp.zeros((NPAD, 16), jnp.float32)
    zeros128 = jnp.zeros((NPAD, D), jnp.float32)

    degp = _deg_kernel(dst_flat, zeros16)
    h1, g1, dis_b = _tc1(x, W1, degp[0, :N], degp[1, :N])

    accp1 = _agg_kernel(src, dst, g1, zeros128)
    h2, g2 = _tc2(accp1[0, :N], accp1[1, :N], dis_b, h1,
                  b1.reshape(1, D), W2)

    accp2 = _agg_kernel(src, dst, g2, zeros128)
    out = _tc3(accp2[0, :N], accp2[1, :N], dis_b, h2,
               b2.reshape(1, D), batch.reshape(1, N), Wfc,
               bfc.reshape(1, D))
    return out


# speedup vs baseline: 30.4324x; 1.0301x over previous
import functools

import jax
import jax.numpy as jnp
from jax import lax
from jax.experimental import pallas as pl
from jax.experimental.pallas import tpu as pltpu
from jax.experimental.pallas import tpu_sc as plsc

N = 10000
E = 320000
D = 128
G = 64
NC = 2
NS = 16
NW = NC * NS
EPW = E // NW
CH = 125
NCHUNK = EPW // CH
SEC = 20
NSEC = NCHUNK // SEC
NPAD = 10112
RPT = NPAD // NS

_mesh = plsc.VectorSubcoreMesh(core_axis_name="c", subcore_axis_name="s")


@functools.partial(
    pl.kernel,
    out_type=jax.ShapeDtypeStruct((NC, NPAD, 16), jnp.float32),
    mesh=_mesh,
    scratch_types=[
        pltpu.VMEM((NCHUNK, CH), jnp.int32),
        pltpu.VMEM((CH, 16), jnp.float32),
        pltpu.VMEM_SHARED((NPAD, 16), jnp.float32),
        pltpu.SemaphoreType.DMA,
    ],
)
def _deg_kernel(dst_hbm, zeros_hbm, out_hbm, didx, ones_v, acc, sem):
    c = lax.axis_index("c")
    s = lax.axis_index("s")
    wid = s * NC + c

    def set_ones(i, carry):
        ones_v[i, :] = jnp.ones((16,), jnp.float32)
        return carry

    lax.fori_loop(0, CH, set_ones, 0)
    pltpu.sync_copy(dst_hbm.at[wid], didx)
    pltpu.sync_copy(zeros_hbm.at[pl.ds(s * RPT, RPT)], acc.at[pl.ds(s * RPT, RPT)])
    plsc.subcore_barrier()

    def body(i, carry):
        pltpu.sync_copy(ones_v, acc.at[didx.at[i]], add=True)
        return carry

    lax.fori_loop(0, NCHUNK, body, 0)
    plsc.subcore_barrier()
    pltpu.sync_copy(acc.at[pl.ds(s * RPT, RPT)], out_hbm.at[c, pl.ds(s * RPT, RPT)])


@functools.partial(
    pl.kernel,
    out_type=jax.ShapeDtypeStruct((NC, NPAD, D), jnp.float32),
    mesh=_mesh,
    scratch_types=[
        pltpu.VMEM((2, SEC, CH), jnp.int32),
        pltpu.VMEM((2, SEC, CH), jnp.int32),
        pltpu.VMEM((2, CH, D), jnp.float32),
        pltpu.VMEM_SHARED((NPAD, D), jnp.float32),
        pltpu.SemaphoreType.DMA,
        pltpu.SemaphoreType.DMA,
        pltpu.SemaphoreType.DMA,
        pltpu.SemaphoreType.DMA,
    ],
)
def _agg_kernel(src_hbm, dst_hbm, g_hbm, zeros_hbm, out_hbm,
                sidx, didx, rows, acc, sem0, sem1, isems, isemd):
    c = lax.axis_index("c")
    s = lax.axis_index("s")
    wid = s * NC + c
    pltpu.sync_copy(src_hbm.at[wid, 0], sidx.at[0])
    pltpu.sync_copy(dst_hbm.at[wid, 0], didx.at[0])
    pltpu.sync_copy(zeros_hbm.at[pl.ds(s * RPT, RPT)], acc.at[pl.ds(s * RPT, RPT)])
    plsc.subcore_barrier()

    pltpu.async_copy(g_hbm.at[sidx.at[0, 0]], rows.at[0], sem0)

    def gather(i, buf, sem):
        sec = i // SEC
        pltpu.async_copy(g_hbm.at[sidx.at[sec % 2, i % SEC]], rows.at[buf], sem)

    def gwait(i, buf, sem):
        sec = i // SEC
        pltpu.make_async_copy(
            g_hbm.at[sidx.at[sec % 2, i % SEC]], rows.at[buf], sem).wait()

    def scatter(i, buf):
        sec = i // SEC
        pltpu.sync_copy(rows.at[buf], acc.at[didx.at[sec % 2, i % SEC]],
                        add=True)

    def body(j, carry):
        i0 = 2 * j
        i1 = i0 + 1
        i2 = i0 + 2
        sec0 = i0 // SEC

        @pl.when((i0 % SEC == 0) & (sec0 + 1 < NSEC))
        def _():
            nxt = sec0 + 1
            pltpu.async_copy(src_hbm.at[wid, nxt], sidx.at[nxt % 2], isems)
            pltpu.async_copy(dst_hbm.at[wid, nxt], didx.at[nxt % 2], isemd)

        @pl.when((i2 < NCHUNK) & (i2 % SEC == 0))
        def _():
            nxt = i2 // SEC
            pltpu.make_async_copy(src_hbm.at[wid, nxt], sidx.at[nxt % 2],
                                  isems).wait()
            pltpu.make_async_copy(dst_hbm.at[wid, nxt], didx.at[nxt % 2],
                                  isemd).wait()

        @pl.when(i1 < NCHUNK)
        def _():
            gather(i1, 1, sem1)

        gwait(i0, 0, sem0)
        scatter(i0, 0)

        @pl.when(i2 < NCHUNK)
        def _():
            gather(i2, 0, sem0)

        @pl.when(i1 < NCHUNK)
        def _():
            gwait(i1, 1, sem1)
            scatter(i1, 1)

        return carry

    lax.fori_loop(0, (NCHUNK + 1) // 2, body, 0)
    plsc.subcore_barrier()
    pltpu.sync_copy(acc.at[pl.ds(s * RPT, RPT)], out_hbm.at[c, pl.ds(s * RPT, RPT)])


def _tc1_body(x_ref, w1_ref, d0_ref, d1_ref, h1_ref, g1_ref, dis_ref):
    deg = 1.0 + d0_ref[:, 0:1] + d1_ref[:, 0:1]
    dis = jnp.broadcast_to(lax.rsqrt(deg), (N, D))
    h1 = jnp.dot(x_ref[:, :], w1_ref[:, :], preferred_element_type=jnp.float32)
    h1_ref[:, :] = h1
    g1_ref[:, :] = dis * h1
    dis_ref[:, :] = dis


def _tc2_body(a0_ref, a1_ref, dis_ref, h1_ref, b1_ref, w2_ref, h2_ref, g2_ref):
    d = dis_ref[:, :]
    h1o = jnp.maximum(
        d * (a0_ref[:, :] + a1_ref[:, :]) + d * d * h1_ref[:, :] + b1_ref[:, :],
        0.0)
    h2 = jnp.dot(h1o, w2_ref[:, :], preferred_element_type=jnp.float32)
    h2_ref[:, :] = h2
    g2_ref[:, :] = d * h2


def _tc3_body(a0_ref, a1_ref, dis_ref, h2_ref, b2_ref, batch_ref, wfc_ref,
              bfc_ref, out_ref):
    d = dis_ref[:, :]
    h2o = jnp.maximum(
        d * (a0_ref[:, :] + a1_ref[:, :]) + d * d * h2_ref[:, :] + b2_ref[:, :],
        0.0)
    gids = lax.broadcasted_iota(jnp.int32, (G, N), 0)
    onehot_t = (gids == batch_ref[:, :]).astype(jnp.float32)
    sums = jnp.dot(onehot_t, h2o, preferred_element_type=jnp.float32)
    cnts = jnp.sum(onehot_t, axis=1, keepdims=True)
    emb = sums / jnp.maximum(cnts, 1.0)
    out_ref[:, :] = (
        jnp.dot(emb, wfc_ref[:, :], preferred_element_type=jnp.float32)
        + bfc_ref[:, :])


_tc1 = pl.pallas_call(
    _tc1_body,
    out_shape=[
        jax.ShapeDtypeStruct((N, D), jnp.float32),
        jax.ShapeDtypeStruct((N, D), jnp.float32),
        jax.ShapeDtypeStruct((N, D), jnp.float32),
    ],
)

_tc2 = pl.pallas_call(
    _tc2_body,
    out_shape=[
        jax.ShapeDtypeStruct((N, D), jnp.float32),
        jax.ShapeDtypeStruct((N, D), jnp.float32),
    ],
)

_tc3 = pl.pallas_call(
    _tc3_body,
    out_shape=jax.ShapeDtypeStruct((G, D), jnp.float32),
)


def kernel(x, edge_index, batch, W1, b1, W2, b2, Wfc, bfc):
    src = edge_index[0].reshape(NW, NSEC, SEC, CH)
    dst = edge_index[1].reshape(NW, NSEC, SEC, CH)
    dst_flat = edge_index[1].reshape(NW, NCHUNK, CH)
    zeros16 = jnp.zeros((NPAD, 16), jnp.float32)
    zeros128 = jnp.zeros((NPAD, D), jnp.float32)

    degp = _deg_kernel(dst_flat, zeros16)
    h1, g1, dis_b = _tc1(x, W1, degp[0, :N], degp[1, :N])

    accp1 = _agg_kernel(src, dst, g1, zeros128)
    h2, g2 = _tc2(accp1[0, :N], accp1[1, :N], dis_b, h1,
                  b1.reshape(1, D), W2)

    accp2 = _agg_kernel(src, dst, g2, zeros128)
    out = _tc3(accp2[0, :N], accp2[1, :N], dis_b, h2,
               b2.reshape(1, D), batch.reshape(1, N), Wfc,
               bfc.reshape(1, D))
    return out


# speedup vs baseline: 30.5397x; 1.0035x over previous
import functools

import jax
import jax.numpy as jnp
from jax import lax
from jax.experimental import pallas as pl
from jax.experimental.pallas import tpu as pltpu
from jax.experimental.pallas import tpu_sc as plsc

N = 10000
E = 320000
D = 128
G = 64
NC = 2
NS = 16
NW = NC * NS
EPW = E // NW
CH = 125
NCHUNK = EPW // CH
SEC = 20
NSEC = NCHUNK // SEC
NPAD = 10112
RPT = NPAD // NS

_mesh = plsc.VectorSubcoreMesh(core_axis_name="c", subcore_axis_name="s")


@functools.partial(
    pl.kernel,
    out_type=jax.ShapeDtypeStruct((NC, NPAD, 16), jnp.float32),
    mesh=_mesh,
    scratch_types=[
        pltpu.VMEM((NCHUNK, CH), jnp.int32),
        pltpu.VMEM((CH, 16), jnp.float32),
        pltpu.VMEM_SHARED((NPAD, 16), jnp.float32),
        pltpu.SemaphoreType.DMA,
    ],
)
def _deg_kernel(dst_hbm, zeros_hbm, out_hbm, didx, ones_v, acc, sem):
    c = lax.axis_index("c")
    s = lax.axis_index("s")
    wid = s * NC + c

    def set_ones(i, carry):
        ones_v[i, :] = jnp.ones((16,), jnp.float32)
        return carry

    lax.fori_loop(0, CH, set_ones, 0)
    pltpu.sync_copy(dst_hbm.at[wid], didx)
    pltpu.sync_copy(zeros_hbm.at[pl.ds(s * RPT, RPT)], acc.at[pl.ds(s * RPT, RPT)])
    plsc.subcore_barrier()

    def body(i, carry):
        pltpu.sync_copy(ones_v, acc.at[didx.at[i]], add=True)
        return carry

    lax.fori_loop(0, NCHUNK, body, 0)
    plsc.subcore_barrier()
    pltpu.sync_copy(acc.at[pl.ds(s * RPT, RPT)], out_hbm.at[c, pl.ds(s * RPT, RPT)])


@functools.partial(
    pl.kernel,
    out_type=jax.ShapeDtypeStruct((NC, NPAD, D), jnp.float32),
    mesh=_mesh,
    scratch_types=[
        pltpu.VMEM((2, SEC, CH), jnp.int32),
        pltpu.VMEM((2, SEC, CH), jnp.int32),
        pltpu.VMEM((2, CH, D), jnp.float32),
        pltpu.VMEM_SHARED((NPAD, D), jnp.float32),
        pltpu.SemaphoreType.DMA,
        pltpu.SemaphoreType.DMA,
        pltpu.SemaphoreType.DMA,
        pltpu.SemaphoreType.DMA,
    ],
)
def _agg_kernel(src_hbm, dst_hbm, g_hbm, zeros_hbm, out_hbm,
                sidx, didx, rows, acc, sem0, sem1, isems, isemd):
    c = lax.axis_index("c")
    s = lax.axis_index("s")
    wid = s * NC + c
    pltpu.sync_copy(src_hbm.at[wid, 0], sidx.at[0])
    pltpu.sync_copy(dst_hbm.at[wid, 0], didx.at[0])
    pltpu.sync_copy(zeros_hbm.at[pl.ds(s * RPT, RPT)], acc.at[pl.ds(s * RPT, RPT)])
    plsc.subcore_barrier()

    pltpu.async_copy(g_hbm.at[sidx.at[0, 0]], rows.at[0], sem0)

    def gather(i, buf, sem):
        sec = i // SEC
        pltpu.async_copy(g_hbm.at[sidx.at[sec % 2, i % SEC]], rows.at[buf], sem)

    def gwait(i, buf, sem):
        sec = i // SEC
        pltpu.make_async_copy(
            g_hbm.at[sidx.at[sec % 2, i % SEC]], rows.at[buf], sem).wait()

    def scatter(i, buf):
        sec = i // SEC
        pltpu.sync_copy(rows.at[buf], acc.at[didx.at[sec % 2, i % SEC]],
                        add=True)

    def body(j, carry):
        i0 = 2 * j
        i1 = i0 + 1
        i2 = i0 + 2
        sec0 = i0 // SEC

        @pl.when((i0 % SEC == 0) & (sec0 + 1 < NSEC))
        def _():
            nxt = sec0 + 1
            pltpu.async_copy(src_hbm.at[wid, nxt], sidx.at[nxt % 2], isems)
            pltpu.async_copy(dst_hbm.at[wid, nxt], didx.at[nxt % 2], isemd)

        @pl.when((i2 < NCHUNK) & (i2 % SEC == 0))
        def _():
            nxt = i2 // SEC
            pltpu.make_async_copy(src_hbm.at[wid, nxt], sidx.at[nxt % 2],
                                  isems).wait()
            pltpu.make_async_copy(dst_hbm.at[wid, nxt], didx.at[nxt % 2],
                                  isemd).wait()

        @pl.when(i1 < NCHUNK)
        def _():
            gather(i1, 1, sem1)

        gwait(i0, 0, sem0)
        scatter(i0, 0)

        @pl.when(i2 < NCHUNK)
        def _():
            gather(i2, 0, sem0)

        @pl.when(i1 < NCHUNK)
        def _():
            gwait(i1, 1, sem1)
            scatter(i1, 1)

        return carry

    lax.fori_loop(0, (NCHUNK + 1) // 2, body, 0)
    plsc.subcore_barrier()
    pltpu.sync_copy(acc.at[pl.ds(s * RPT, RPT)], out_hbm.at[c, pl.ds(s * RPT, RPT)])


def _tc0_body(x_ref, w1_ref, h1_ref):
    h1_ref[:, :] = jnp.dot(x_ref[:, :], w1_ref[:, :],
                           preferred_element_type=jnp.float32)


def _tc1_body(h1_ref, d0_ref, d1_ref, g1_ref, dis_ref):
    deg = 1.0 + d0_ref[:, 0:1] + d1_ref[:, 0:1]
    dis = jnp.broadcast_to(lax.rsqrt(deg), (N, D))
    g1_ref[:, :] = dis * h1_ref[:, :]
    dis_ref[:, :] = dis


def _tc2_body(a0_ref, a1_ref, dis_ref, h1_ref, b1_ref, w2_ref, h2_ref, g2_ref):
    d = dis_ref[:, :]
    h1o = jnp.maximum(
        d * (a0_ref[:, :] + a1_ref[:, :]) + d * d * h1_ref[:, :] + b1_ref[:, :],
        0.0)
    h2 = jnp.dot(h1o, w2_ref[:, :], preferred_element_type=jnp.float32)
    h2_ref[:, :] = h2
    g2_ref[:, :] = d * h2


def _tc3_body(a0_ref, a1_ref, dis_ref, h2_ref, b2_ref, batch_ref, wfc_ref,
              bfc_ref, out_ref):
    d = dis_ref[:, :]
    h2o = jnp.maximum(
        d * (a0_ref[:, :] + a1_ref[:, :]) + d * d * h2_ref[:, :] + b2_ref[:, :],
        0.0)
    gids = lax.broadcasted_iota(jnp.int32, (G, N), 0)
    onehot_t = (gids == batch_ref[:, :]).astype(jnp.float32)
    sums = jnp.dot(onehot_t, h2o, preferred_element_type=jnp.float32)
    cnts = jnp.sum(onehot_t, axis=1, keepdims=True)
    emb = sums / jnp.maximum(cnts, 1.0)
    out_ref[:, :] = (
        jnp.dot(emb, wfc_ref[:, :], preferred_element_type=jnp.float32)
        + bfc_ref[:, :])


_tc0 = pl.pallas_call(
    _tc0_body,
    out_shape=jax.ShapeDtypeStruct((N, D), jnp.float32),
)

_tc1 = pl.pallas_call(
    _tc1_body,
    out_shape=[
        jax.ShapeDtypeStruct((N, D), jnp.float32),
        jax.ShapeDtypeStruct((N, D), jnp.float32),
    ],
)

_tc2 = pl.pallas_call(
    _tc2_body,
    out_shape=[
        jax.ShapeDtypeStruct((N, D), jnp.float32),
        jax.ShapeDtypeStruct((N, D), jnp.float32),
    ],
)

_tc3 = pl.pallas_call(
    _tc3_body,
    out_shape=jax.ShapeDtypeStruct((G, D), jnp.float32),
)


def kernel(x, edge_index, batch, W1, b1, W2, b2, Wfc, bfc):
    src = edge_index[0].reshape(NW, NSEC, SEC, CH)
    dst = edge_index[1].reshape(NW, NSEC, SEC, CH)
    dst_flat = edge_index[1].reshape(NW, NCHUNK, CH)
    zeros16 = jnp.zeros((NPAD, 16), jnp.float32)
    zeros128 = jnp.zeros((NPAD, D), jnp.float32)

    h1 = _tc0(x, W1)
    degp = _deg_kernel(dst_flat, zeros16)
    g1, dis_b = _tc1(h1, degp[0, :N], degp[1, :N])

    accp1 = _agg_kernel(src, dst, g1, zeros128)
    h2, g2 = _tc2(accp1[0, :N], accp1[1, :N], dis_b, h1,
                  b1.reshape(1, D), W2)

    accp2 = _agg_kernel(src, dst, g2, zeros128)
    out = _tc3(accp2[0, :N], accp2[1, :N], dis_b, h2,
               b2.reshape(1, D), batch.reshape(1, N), Wfc,
               bfc.reshape(1, D))
    return out
